# baseline (speedup 1.0000x reference)
import functools

import jax
import jax.numpy as jnp
from jax import lax
from jax.experimental import pallas as pl
from jax.experimental.pallas import tpu as pltpu
from jax.experimental.pallas import tpu_sc as plsc

N = 16384
D = 4
NUM_TILES = 16
ROWS_PER_TILE = N // NUM_TILES
ELEMS_PER_TILE = ROWS_PER_TILE * D
VECS_PER_TILE = ELEMS_PER_TILE // 16
LBL_VECS_PER_TILE = ROWS_PER_TILE // 16


def _sc_body(a_hbm, b_hbm, lbl_hbm, out_hbm,
             a_v, b_v, l_v, res_v, all_v, out_v, shared):
    sid = lax.axis_index("s")

    pltpu.sync_copy(a_hbm.at[pl.ds(sid * ROWS_PER_TILE, ROWS_PER_TILE)], a_v)
    pltpu.sync_copy(b_hbm.at[pl.ds(sid * ROWS_PER_TILE, ROWS_PER_TILE)], b_v)
    pltpu.sync_copy(lbl_hbm.at[pl.ds(sid * ROWS_PER_TILE, ROWS_PER_TILE)], l_v)

    iota = lax.broadcasted_iota(jnp.int32, (16,), 0)
    quad = iota >> 2
    lane3 = iota & 3

    def mse_body(v, acc):
        ridx = quad + v * 4
        a = plsc.load_gather(a_v, [ridx, lane3])
        b = plsc.load_gather(b_v, [ridx, lane3])
        d = a - b
        m = plsc.load_gather(l_v, [ridx])
        return acc + d * d * m.astype(jnp.float32)

    acc = lax.fori_loop(0, VECS_PER_TILE, mse_body,
                        jnp.zeros((16,), jnp.float32))

    def cnt_body(k, c):
        return c + l_v[pl.ds(k * 16, 16)].astype(jnp.float32)

    cnt = lax.fori_loop(0, LBL_VECS_PER_TILE, cnt_body,
                        jnp.zeros((16,), jnp.float32))

    res_v[0] = acc
    res_v[1] = cnt
    pltpu.sync_copy(res_v, shared.at[sid])
    plsc.subcore_barrier()

    @pl.when(sid == 0)
    def _():
        pltpu.sync_copy(shared, all_v)

        def red_body(i, carry):
            ts, tc = carry
            return ts + all_v[i, 0], tc + all_v[i, 1]

        ts, tc = lax.fori_loop(0, NUM_TILES, red_body,
                               (jnp.zeros((16,), jnp.float32),
                                jnp.zeros((16,), jnp.float32)))
        s_vec = jnp.full((16,), jnp.sum(ts), dtype=jnp.float32)
        c_vec = jnp.full((16,), jnp.sum(tc), dtype=jnp.float32)
        out_v[...] = s_vec / c_vec
        pltpu.sync_copy(out_v, out_hbm)


@jax.jit
def _bbox_loss(a, b, label):
    mesh = plsc.VectorSubcoreMesh(core_axis_name="c", subcore_axis_name="s",
                                  num_cores=1)
    call = functools.partial(
        pl.kernel,
        out_type=jax.ShapeDtypeStruct((16,), jnp.float32),
        mesh=mesh,
        compiler_params=pltpu.CompilerParams(needs_layout_passes=False,
                                             use_tc_tiling_on_sc=False),
        scratch_types=[
            pltpu.VMEM((ROWS_PER_TILE, D), jnp.float32),
            pltpu.VMEM((ROWS_PER_TILE, D), jnp.float32),
            pltpu.VMEM((ROWS_PER_TILE,), jnp.int32),
            pltpu.VMEM((2, 16), jnp.float32),
            pltpu.VMEM((NUM_TILES, 2, 16), jnp.float32),
            pltpu.VMEM((16,), jnp.float32),
            pltpu.VMEM_SHARED((NUM_TILES, 2, 16), jnp.float32),
        ],
    )(_sc_body)
    out = call(a, b, label)
    return out[0]


def kernel(bbox_out, bbox_target, label):
    return _bbox_loss(bbox_out, bbox_target, label)


# speedup vs baseline: 2.3037x; 2.3037x over previous
import functools

import jax
import jax.numpy as jnp
from jax import lax
from jax.experimental import pallas as pl
from jax.experimental.pallas import tpu as pltpu
from jax.experimental.pallas import tpu_sc as plsc

N = 16384
D = 4
NUM_TILES = 16
ROWS_PER_TILE = N // NUM_TILES
VECS_PER_TILE = ROWS_PER_TILE // 16


def _sc_body(a_hbm, b_hbm, lbl_hbm, out_hbm,
             a_v, b_v, l_v, res_v, all_v, out_v, shared):
    sid = lax.axis_index("s")
    r0 = sid * ROWS_PER_TILE

    for p in range(D):
        pltpu.sync_copy(a_hbm.at[p, pl.ds(r0, ROWS_PER_TILE)], a_v.at[p])
        pltpu.sync_copy(b_hbm.at[p, pl.ds(r0, ROWS_PER_TILE)], b_v.at[p])
    pltpu.sync_copy(lbl_hbm.at[pl.ds(r0, ROWS_PER_TILE)], l_v)

    def body(k, carry):
        a0, a1, a2, a3, cnt = carry
        mf = l_v[pl.ds(k * 16, 16)].astype(jnp.float32)
        s = pl.ds(k * 16, 16)
        d0 = a_v[0, s] - b_v[0, s]
        d1 = a_v[1, s] - b_v[1, s]
        d2 = a_v[2, s] - b_v[2, s]
        d3 = a_v[3, s] - b_v[3, s]
        return (a0 + d0 * d0 * mf, a1 + d1 * d1 * mf,
                a2 + d2 * d2 * mf, a3 + d3 * d3 * mf, cnt + mf)

    z = jnp.zeros((16,), jnp.float32)
    a0, a1, a2, a3, cnt = lax.fori_loop(0, VECS_PER_TILE, body,
                                        (z, z, z, z, z), unroll=4)
    acc = (a0 + a1) + (a2 + a3)

    res_v[0] = acc
    res_v[1] = cnt
    pltpu.sync_copy(res_v, shared.at[sid])
    plsc.subcore_barrier()

    @pl.when(sid == 0)
    def _():
        pltpu.sync_copy(shared, all_v)

        def red_body(i, carry):
            ts, tc = carry
            return ts + all_v[i, 0], tc + all_v[i, 1]

        ts, tc = lax.fori_loop(0, NUM_TILES, red_body, (z, z))
        s_vec = jnp.full((16,), jnp.sum(ts), dtype=jnp.float32)
        c_vec = jnp.full((16,), jnp.sum(tc), dtype=jnp.float32)
        out_v[...] = s_vec / c_vec
        pltpu.sync_copy(out_v, out_hbm)


@jax.jit
def _bbox_loss(a, b, label):
    mesh = plsc.VectorSubcoreMesh(core_axis_name="c", subcore_axis_name="s",
                                  num_cores=1)
    call = functools.partial(
        pl.kernel,
        out_type=jax.ShapeDtypeStruct((16,), jnp.float32),
        mesh=mesh,
        compiler_params=pltpu.CompilerParams(needs_layout_passes=False,
                                             use_tc_tiling_on_sc=False),
        scratch_types=[
            pltpu.VMEM((D, ROWS_PER_TILE), jnp.float32),
            pltpu.VMEM((D, ROWS_PER_TILE), jnp.float32),
            pltpu.VMEM((ROWS_PER_TILE,), jnp.int32),
            pltpu.VMEM((2, 16), jnp.float32),
            pltpu.VMEM((NUM_TILES, 2, 16), jnp.float32),
            pltpu.VMEM((16,), jnp.float32),
            pltpu.VMEM_SHARED((NUM_TILES, 2, 16), jnp.float32),
        ],
    )(_sc_body)
    out = call(a, b, label)
    return out[0]


def kernel(bbox_out, bbox_target, label):
    return _bbox_loss(bbox_out.T, bbox_target.T, label)


# speedup vs baseline: 2.6940x; 1.1694x over previous
import functools

import jax
import jax.numpy as jnp
from jax import lax
from jax.experimental import pallas as pl
from jax.experimental.pallas import tpu as pltpu
from jax.experimental.pallas import tpu_sc as plsc

N = 16384
D = 4
NUM_TILES = 16
ROWS_PER_TILE = N // NUM_TILES
VECS_PER_TILE = ROWS_PER_TILE // 16


def _sc_body(a_hbm, b_hbm, lbl_hbm, out_hbm,
             a_v, b_v, l_v, res_v, all_v, out_v, shared):
    sid = lax.axis_index("s")
    r0 = sid * ROWS_PER_TILE

    pltpu.sync_copy(a_hbm.at[:, pl.ds(r0, ROWS_PER_TILE)], a_v)
    pltpu.sync_copy(b_hbm.at[:, pl.ds(r0, ROWS_PER_TILE)], b_v)
    pltpu.sync_copy(lbl_hbm.at[pl.ds(r0, ROWS_PER_TILE)], l_v)

    def body(k, carry):
        a0, a1, a2, a3, cnt = carry
        mf = l_v[pl.ds(k * 16, 16)].astype(jnp.float32)
        s = pl.ds(k * 16, 16)
        d0 = a_v[0, s] - b_v[0, s]
        d1 = a_v[1, s] - b_v[1, s]
        d2 = a_v[2, s] - b_v[2, s]
        d3 = a_v[3, s] - b_v[3, s]
        return (a0 + d0 * d0 * mf, a1 + d1 * d1 * mf,
                a2 + d2 * d2 * mf, a3 + d3 * d3 * mf, cnt + mf)

    z = jnp.zeros((16,), jnp.float32)
    a0, a1, a2, a3, cnt = lax.fori_loop(0, VECS_PER_TILE, body,
                                        (z, z, z, z, z))
    acc = (a0 + a1) + (a2 + a3)

    res_v[0] = acc
    res_v[1] = cnt
    pltpu.sync_copy(res_v, shared.at[sid])
    plsc.subcore_barrier()

    @pl.when(sid == 0)
    def _():
        pltpu.sync_copy(shared, all_v)

        def red_body(i, carry):
            ts, tc = carry
            return ts + all_v[i, 0], tc + all_v[i, 1]

        ts, tc = lax.fori_loop(0, NUM_TILES, red_body, (z, z))
        s_vec = jnp.full((16,), jnp.sum(ts), dtype=jnp.float32)
        c_vec = jnp.full((16,), jnp.sum(tc), dtype=jnp.float32)
        out_v[...] = s_vec / c_vec
        pltpu.sync_copy(out_v, out_hbm)


@jax.jit
def _bbox_loss(a, b, label):
    mesh = plsc.VectorSubcoreMesh(core_axis_name="c", subcore_axis_name="s",
                                  num_cores=1)
    call = functools.partial(
        pl.kernel,
        out_type=jax.ShapeDtypeStruct((16,), jnp.float32),
        mesh=mesh,
        compiler_params=pltpu.CompilerParams(needs_layout_passes=False,
                                             use_tc_tiling_on_sc=False),
        scratch_types=[
            pltpu.VMEM((D, ROWS_PER_TILE), jnp.float32),
            pltpu.VMEM((D, ROWS_PER_TILE), jnp.float32),
            pltpu.VMEM((ROWS_PER_TILE,), jnp.int32),
            pltpu.VMEM((2, 16), jnp.float32),
            pltpu.VMEM((NUM_TILES, 2, 16), jnp.float32),
            pltpu.VMEM((16,), jnp.float32),
            pltpu.VMEM_SHARED((NUM_TILES, 2, 16), jnp.float32),
        ],
    )(_sc_body)
    out = call(a, b, label)
    return out[0]


def kernel(bbox_out, bbox_target, label):
    return _bbox_loss(bbox_out.T, bbox_target.T, label)


# speedup vs baseline: 2.6984x; 1.0016x over previous
import functools

import jax
import jax.numpy as jnp
from jax import lax
from jax.experimental import pallas as pl
from jax.experimental.pallas import tpu as pltpu
from jax.experimental.pallas import tpu_sc as plsc

N = 16384
D = 4
NUM_TILES = 16
ROWS_PER_TILE = N // NUM_TILES
VECS_PER_TILE = ROWS_PER_TILE // 16


def _sc_body(a_hbm, b_hbm, lbl_hbm, out_hbm,
             a_v, b_v, l_v, res_v, all_v, out_v, shared):
    sid = lax.axis_index("s")
    r0 = sid * ROWS_PER_TILE

    pltpu.sync_copy(a_hbm.at[:, pl.ds(r0, ROWS_PER_TILE)], a_v)
    pltpu.sync_copy(b_hbm.at[:, pl.ds(r0, ROWS_PER_TILE)], b_v)
    pltpu.sync_copy(lbl_hbm.at[pl.ds(r0, ROWS_PER_TILE)], l_v)

    def body(k, carry):
        a0, a1, a2, a3, cnt = carry
        mf = l_v[pl.ds(k * 16, 16)].astype(jnp.float32)
        s = pl.ds(k * 16, 16)
        d0 = a_v[0, s] - b_v[0, s]
        d1 = a_v[1, s] - b_v[1, s]
        d2 = a_v[2, s] - b_v[2, s]
        d3 = a_v[3, s] - b_v[3, s]
        return (a0 + d0 * d0 * mf, a1 + d1 * d1 * mf,
                a2 + d2 * d2 * mf, a3 + d3 * d3 * mf, cnt + mf)

    z = jnp.zeros((16,), jnp.float32)
    a0, a1, a2, a3, cnt = lax.fori_loop(0, VECS_PER_TILE, body,
                                        (z, z, z, z, z))
    acc = (a0 + a1) + (a2 + a3)

    res_v[0] = acc
    res_v[1] = cnt
    pltpu.sync_copy(res_v, shared.at[sid])
    plsc.subcore_barrier()

    @pl.when(sid == 0)
    def _():
        pltpu.sync_copy(shared, all_v)

        def red_body(i, carry):
            ts, tc = carry
            return ts + all_v[i, 0], tc + all_v[i, 1]

        ts, tc = lax.fori_loop(0, NUM_TILES, red_body, (z, z))
        s_vec = jnp.full((16,), jnp.sum(ts), dtype=jnp.float32)
        c_vec = jnp.full((16,), jnp.sum(tc), dtype=jnp.float32)
        out_v[...] = s_vec / c_vec
        pltpu.sync_copy(out_v, out_hbm)


@jax.jit
def _bbox_loss(a, b, label):
    mesh = plsc.VectorSubcoreMesh(core_axis_name="c", subcore_axis_name="s",
                                  num_cores=1)
    call = functools.partial(
        pl.kernel,
        out_type=jax.ShapeDtypeStruct((16,), jnp.float32),
        mesh=mesh,
        compiler_params=pltpu.CompilerParams(needs_layout_passes=False,
                                             use_tc_tiling_on_sc=False,
                                             skip_device_barrier=True),
        scratch_types=[
            pltpu.VMEM((D, ROWS_PER_TILE), jnp.float32),
            pltpu.VMEM((D, ROWS_PER_TILE), jnp.float32),
            pltpu.VMEM((ROWS_PER_TILE,), jnp.int32),
            pltpu.VMEM((2, 16), jnp.float32),
            pltpu.VMEM((NUM_TILES, 2, 16), jnp.float32),
            pltpu.VMEM((16,), jnp.float32),
            pltpu.VMEM_SHARED((NUM_TILES, 2, 16), jnp.float32),
        ],
    )(_sc_body)
    out = call(a, b, label)
    return out[0]


def kernel(bbox_out, bbox_target, label):
    return _bbox_loss(bbox_out.T, bbox_target.T, label)
